# baseline (speedup 1.0000x reference)
import functools

import jax
import jax.numpy as jnp
from jax import lax
from jax.experimental import pallas as pl
from jax.experimental.pallas import tpu as pltpu
from jax.experimental.pallas import tpu_sc as plsc

_NC = 2
_NS = 16
_NW = _NC * _NS

_N = 8192 * 4096
_PER_TILE = _N // _NW
_CH = 32768
_CHUNKS = _PER_TILE // _CH
_PAIRS = _CHUNKS // 2
_GROUPS = _CH // 64
_NY = 4096
_NWORDS = 1 << 15


def _isin_body(x_hbm, y_hbm, out_hbm, ybuf, bitmap,
               xb0, xb1, ob0, ob1, isem0, isem1, osem0, osem1):
    wid = lax.axis_index("s") * _NC + lax.axis_index("c")
    base = wid * _PER_TILE
    obase = wid * (_PER_TILE // 4)

    pltpu.make_async_copy(x_hbm.at[pl.ds(base, _CH)], xb0, isem0).start()
    pltpu.make_async_copy(x_hbm.at[pl.ds(base + _CH, _CH)], xb1, isem1).start()

    pltpu.sync_copy(y_hbm, ybuf)

    zeros = jnp.zeros((16,), jnp.int32)

    def zbody(i, c):
        bitmap[pl.ds(i * 16, 16)] = zeros
        return c

    lax.fori_loop(0, _NWORDS // 16, zbody, 0, unroll=4)

    iota4 = lax.broadcasted_iota(jnp.int32, (16,), 0) * 4
    one = jnp.full((16,), 1, jnp.int32)

    def bbody(i, c):
        yv = ybuf[pl.ds(i * 16, 16)]
        w = lax.shift_right_logical(yv, 5)
        sh = yv & 31
        bit = one << sh

        def round_fn(_, c2):
            cur = plsc.load_gather(bitmap, [w])
            miss = (lax.shift_right_logical(cur, sh) & one) == 0
            plsc.store_scatter(bitmap, [w], cur | bit, mask=miss)
            return c2

        round_fn(0, 0)
        cur = plsc.load_gather(bitmap, [w])
        miss = (lax.shift_right_logical(cur, sh) & one) == 0
        cnt = plsc.all_reduce_population_count(miss)

        @pl.when(cnt[0] > 0)
        def _():
            lax.fori_loop(0, 15, round_fn, 0)

        return c

    lax.fori_loop(0, _NY // 16, bbody, 0)

    def compute_chunk(xb, ob):
        def gbody(g, c):
            wr = g >> 8
            c0 = (g & 255) * 16
            off = wr * 16384 + c0
            acc = None
            for k in range(4):
                xk = xb[pl.ds(off + k * 4096, 16)]
                word = plsc.load_gather(bitmap, [lax.shift_right_logical(xk, 5)])
                r = lax.shift_right_logical(word, xk & 31) & one
                acc = r if k == 0 else acc | (r << (8 * k))
            ob[wr, pl.ds(c0, 16)] = acc
            return c

        lax.fori_loop(0, _GROUPS, gbody, 0, unroll=4)

    bufs = ((xb0, ob0, isem0, osem0), (xb1, ob1, isem1, osem1))
    _ROWS = _CH // 4096
    row_base = wid * (_PER_TILE // 4096)

    def pair_body(p, c):
        for b in range(2):
            xb, ob, isem, osem = bufs[b]
            ci = p * 2 + b
            off = base + ci * _CH
            pltpu.make_async_copy(x_hbm.at[pl.ds(off, _CH)], xb, isem).wait()

            @pl.when(ci >= 2)
            def _():
                prev = row_base + (ci - 2) * _ROWS
                pltpu.make_async_copy(
                    ob.bitcast(jnp.int8),
                    out_hbm.at[pl.ds(prev, _ROWS), :], osem).wait()

            compute_chunk(xb, ob)
            pltpu.make_async_copy(
                ob.bitcast(jnp.int8),
                out_hbm.at[pl.ds(row_base + ci * _ROWS, _ROWS), :],
                osem).start()

            @pl.when(p < _PAIRS - 1)
            def _():
                nxt = base + (ci + 2) * _CH
                pltpu.make_async_copy(x_hbm.at[pl.ds(nxt, _CH)], xb, isem).start()
        return c

    lax.fori_loop(0, _PAIRS, pair_body, 0)

    for b in range(2):
        xb, ob, isem, osem = bufs[b]
        pltpu.make_async_copy(
            ob.bitcast(jnp.int8),
            out_hbm.at[pl.ds(row_base + (_CHUNKS - 2 + b) * _ROWS, _ROWS), :],
            osem).wait()


_isin_sc = functools.partial(
    pl.kernel,
    out_type=jax.ShapeDtypeStruct((8192, 4096), jnp.int8),
    mesh=plsc.VectorSubcoreMesh(core_axis_name="c", subcore_axis_name="s"),
    scratch_types=[
        pltpu.VMEM((_NY,), jnp.int32),
        pltpu.VMEM((_NWORDS,), jnp.int32),
        pltpu.VMEM((_CH,), jnp.int32),
        pltpu.VMEM((_CH,), jnp.int32),
        pltpu.VMEM((_CH // 16384, 4096), jnp.int32),
        pltpu.VMEM((_CH // 16384, 4096), jnp.int32),
        pltpu.SemaphoreType.DMA,
        pltpu.SemaphoreType.DMA,
        pltpu.SemaphoreType.DMA,
        pltpu.SemaphoreType.DMA,
    ],
    compiler_params=pltpu.CompilerParams(needs_layout_passes=False),
)(_isin_body)


def kernel(x, y):
    xf = x.ravel().astype(jnp.int32)
    return _isin_sc(xf, y.astype(jnp.int32)).astype(jnp.bool_)


# speedup vs baseline: 1.3455x; 1.3455x over previous
import functools

import jax
import jax.numpy as jnp
from jax import lax
from jax.experimental import pallas as pl
from jax.experimental.pallas import tpu as pltpu
from jax.experimental.pallas import tpu_sc as plsc

_NC = 2
_NS = 16
_NW = _NC * _NS

_N = 8192 * 4096
_PER_TILE = _N // _NW
_CH = 32768
_CHUNKS = _PER_TILE // _CH
_PAIRS = _CHUNKS // 2
_GROUPS = _CH // 64
_NY = 4096
_NWORDS = 1 << 15


def _isin_body(x_hbm, y_hbm, out_hbm, ybuf, bitmap,
               xb0, xb1, ob0, ob1, isem0, isem1, osem0, osem1):
    wid = lax.axis_index("s") * _NC + lax.axis_index("c")
    base = wid * _PER_TILE
    obase = wid * (_PER_TILE // 4)

    xrow = wid * (_PER_TILE // 4096)
    pltpu.make_async_copy(x_hbm.at[pl.ds(xrow, 8), :], xb0, isem0).start()
    pltpu.make_async_copy(x_hbm.at[pl.ds(xrow + 8, 8), :], xb1, isem1).start()

    pltpu.sync_copy(y_hbm, ybuf)

    zeros = jnp.zeros((16,), jnp.int32)

    def zbody(i, c):
        bitmap[pl.ds(i * 16, 16)] = zeros
        return c

    lax.fori_loop(0, _NWORDS // 16, zbody, 0, unroll=4)

    iota4 = lax.broadcasted_iota(jnp.int32, (16,), 0) * 4
    one = jnp.full((16,), 1, jnp.int32)

    def bbody(i, c):
        yv = ybuf[pl.ds(i * 16, 16)]
        w = lax.shift_right_logical(yv, 5)
        sh = yv & 31
        bit = one << sh

        def round_fn(_, c2):
            cur = plsc.load_gather(bitmap, [w])
            miss = (lax.shift_right_logical(cur, sh) & one) == 0
            plsc.store_scatter(bitmap, [w], cur | bit, mask=miss)
            return c2

        round_fn(0, 0)
        cur = plsc.load_gather(bitmap, [w])
        miss = (lax.shift_right_logical(cur, sh) & one) == 0
        cnt = plsc.all_reduce_population_count(miss)

        @pl.when(cnt[0] > 0)
        def _():
            lax.fori_loop(0, 15, round_fn, 0)

        return c

    lax.fori_loop(0, _NY // 16, bbody, 0)

    def compute_chunk(xb, ob):
        def gbody(g, c):
            wr = g >> 8
            c0 = (g & 255) * 16
            acc = None
            for k in range(4):
                xk = xb[wr * 4 + k, pl.ds(c0, 16)]
                word = plsc.load_gather(bitmap, [lax.shift_right_logical(xk, 5)])
                r = lax.shift_right_logical(word, xk & 31) & one
                acc = r if k == 0 else acc | (r << (8 * k))
            ob[wr, pl.ds(c0, 16)] = acc
            return c

        lax.fori_loop(0, _GROUPS, gbody, 0)

    bufs = ((xb0, ob0, isem0, osem0), (xb1, ob1, isem1, osem1))
    _ROWS = _CH // 4096
    row_base = wid * (_PER_TILE // 4096)

    def pair_body(p, c):
        for b in range(2):
            xb, ob, isem, osem = bufs[b]
            ci = p * 2 + b
            pltpu.make_async_copy(
                x_hbm.at[pl.ds(xrow + ci * 8, 8), :], xb, isem).wait()

            @pl.when(ci >= 2)
            def _():
                prev = row_base + (ci - 2) * _ROWS
                pltpu.make_async_copy(
                    ob.bitcast(jnp.int8),
                    out_hbm.at[pl.ds(prev, _ROWS), :], osem).wait()

            compute_chunk(xb, ob)
            pltpu.make_async_copy(
                ob.bitcast(jnp.int8),
                out_hbm.at[pl.ds(row_base + ci * _ROWS, _ROWS), :],
                osem).start()

            @pl.when(p < _PAIRS - 1)
            def _():
                nxt = xrow + (ci + 2) * 8
                pltpu.make_async_copy(
                    x_hbm.at[pl.ds(nxt, 8), :], xb, isem).start()
        return c

    lax.fori_loop(0, _PAIRS, pair_body, 0)

    for b in range(2):
        xb, ob, isem, osem = bufs[b]
        pltpu.make_async_copy(
            ob.bitcast(jnp.int8),
            out_hbm.at[pl.ds(row_base + (_CHUNKS - 2 + b) * _ROWS, _ROWS), :],
            osem).wait()


_isin_sc = functools.partial(
    pl.kernel,
    out_type=jax.ShapeDtypeStruct((8192, 4096), jnp.int8),
    mesh=plsc.VectorSubcoreMesh(core_axis_name="c", subcore_axis_name="s"),
    scratch_types=[
        pltpu.VMEM((_NY,), jnp.int32),
        pltpu.VMEM((_NWORDS,), jnp.int32),
        pltpu.VMEM((8, 4096), jnp.int32),
        pltpu.VMEM((8, 4096), jnp.int32),
        pltpu.VMEM((_CH // 16384, 4096), jnp.int32),
        pltpu.VMEM((_CH // 16384, 4096), jnp.int32),
        pltpu.SemaphoreType.DMA,
        pltpu.SemaphoreType.DMA,
        pltpu.SemaphoreType.DMA,
        pltpu.SemaphoreType.DMA,
    ],
    compiler_params=pltpu.CompilerParams(needs_layout_passes=False),
)(_isin_body)


def kernel(x, y):
    return _isin_sc(x.astype(jnp.int32), y.astype(jnp.int32)).astype(jnp.bool_)


# speedup vs baseline: 2.7046x; 2.0101x over previous
import functools

import jax
import jax.numpy as jnp
from jax import lax
from jax.experimental import pallas as pl
from jax.experimental.pallas import tpu as pltpu
from jax.experimental.pallas import tpu_sc as plsc

_NC = 2
_NS = 16
_NW = _NC * _NS

_N = 8192 * 4096
_PER_TILE = _N // _NW
_CH = 32768
_CHUNKS = _PER_TILE // _CH
_PAIRS = _CHUNKS // 2
_GROUPS = _CH // 64
_NY = 4096
_NWORDS = 1 << 15


def _isin_body(x_hbm, y_hbm, out_hbm, ybuf, bitmap,
               xb0, xb1, ob0, ob1, isem0, isem1, osem0, osem1):
    wid = lax.axis_index("s") * _NC + lax.axis_index("c")
    base = wid * _PER_TILE
    obase = wid * (_PER_TILE // 4)

    xrow = wid * (_PER_TILE // 4096)
    pltpu.make_async_copy(x_hbm.at[pl.ds(xrow, 8), :], xb0, isem0).start()
    pltpu.make_async_copy(x_hbm.at[pl.ds(xrow + 8, 8), :], xb1, isem1).start()

    pltpu.sync_copy(y_hbm, ybuf)

    zeros = jnp.zeros((16,), jnp.int32)

    def zbody(i, c):
        bitmap[pl.ds(i * 16, 16)] = zeros
        return c

    lax.fori_loop(0, _NWORDS // 16, zbody, 0, unroll=4)

    iota4 = lax.broadcasted_iota(jnp.int32, (16,), 0) * 4
    one = jnp.full((16,), 1, jnp.int32)

    def bbody(i, c):
        yv = ybuf[pl.ds(i * 16, 16)]
        w = lax.shift_right_logical(yv, 5)
        sh = yv & 31
        bit = one << sh

        def round_fn(_, c2):
            cur = plsc.load_gather(bitmap, [w])
            miss = (lax.shift_right_logical(cur, sh) & one) == 0
            plsc.store_scatter(bitmap, [w], cur | bit, mask=miss)
            return c2

        round_fn(0, 0)
        cur = plsc.load_gather(bitmap, [w])
        miss = (lax.shift_right_logical(cur, sh) & one) == 0
        cnt = plsc.all_reduce_population_count(miss)

        @pl.when(cnt[0] > 0)
        def _():
            lax.fori_loop(0, 15, round_fn, 0)

        return c

    lax.fori_loop(0, _NY // 16, bbody, 0)

    def compute_chunk(xb, ob):
        @plsc.parallel_loop(0, _GROUPS, unroll=4)
        def gbody(g):
            wr = g >> 8
            c0 = (g & 255) * 16
            acc = None
            for k in range(4):
                xk = xb[wr * 4 + k, pl.ds(c0, 16)]
                word = plsc.load_gather(bitmap, [lax.shift_right_logical(xk, 5)])
                r = lax.shift_right_logical(word, xk & 31) & one
                acc = r if k == 0 else acc | (r << (8 * k))
            ob[wr, pl.ds(c0, 16)] = acc

    bufs = ((xb0, ob0, isem0, osem0), (xb1, ob1, isem1, osem1))
    _ROWS = _CH // 4096
    row_base = wid * (_PER_TILE // 4096)

    def pair_body(p, c):
        for b in range(2):
            xb, ob, isem, osem = bufs[b]
            ci = p * 2 + b
            pltpu.make_async_copy(
                x_hbm.at[pl.ds(xrow + ci * 8, 8), :], xb, isem).wait()

            @pl.when(ci >= 2)
            def _():
                prev = row_base + (ci - 2) * _ROWS
                pltpu.make_async_copy(
                    ob.bitcast(jnp.int8),
                    out_hbm.at[pl.ds(prev, _ROWS), :], osem).wait()

            compute_chunk(xb, ob)
            pltpu.make_async_copy(
                ob.bitcast(jnp.int8),
                out_hbm.at[pl.ds(row_base + ci * _ROWS, _ROWS), :],
                osem).start()

            @pl.when(p < _PAIRS - 1)
            def _():
                nxt = xrow + (ci + 2) * 8
                pltpu.make_async_copy(
                    x_hbm.at[pl.ds(nxt, 8), :], xb, isem).start()
        return c

    lax.fori_loop(0, _PAIRS, pair_body, 0)

    for b in range(2):
        xb, ob, isem, osem = bufs[b]
        pltpu.make_async_copy(
            ob.bitcast(jnp.int8),
            out_hbm.at[pl.ds(row_base + (_CHUNKS - 2 + b) * _ROWS, _ROWS), :],
            osem).wait()


_isin_sc = functools.partial(
    pl.kernel,
    out_type=jax.ShapeDtypeStruct((8192, 4096), jnp.int8),
    mesh=plsc.VectorSubcoreMesh(core_axis_name="c", subcore_axis_name="s"),
    scratch_types=[
        pltpu.VMEM((_NY,), jnp.int32),
        pltpu.VMEM((_NWORDS,), jnp.int32),
        pltpu.VMEM((8, 4096), jnp.int32),
        pltpu.VMEM((8, 4096), jnp.int32),
        pltpu.VMEM((_CH // 16384, 4096), jnp.int32),
        pltpu.VMEM((_CH // 16384, 4096), jnp.int32),
        pltpu.SemaphoreType.DMA,
        pltpu.SemaphoreType.DMA,
        pltpu.SemaphoreType.DMA,
        pltpu.SemaphoreType.DMA,
    ],
    compiler_params=pltpu.CompilerParams(needs_layout_passes=False),
)(_isin_body)


def kernel(x, y):
    return _isin_sc(x.astype(jnp.int32), y.astype(jnp.int32)).astype(jnp.bool_)


# speedup vs baseline: 2.7213x; 1.0062x over previous
import functools

import jax
import jax.numpy as jnp
from jax import lax
from jax.experimental import pallas as pl
from jax.experimental.pallas import tpu as pltpu
from jax.experimental.pallas import tpu_sc as plsc

_NC = 2
_NS = 16
_NW = _NC * _NS

_N = 8192 * 4096
_PER_TILE = _N // _NW
_CH = 32768
_CHUNKS = _PER_TILE // _CH
_PAIRS = _CHUNKS // 2
_GROUPS = _CH // 64
_NY = 4096
_NWORDS = 1 << 15


def _isin_body(x_hbm, y_hbm, out_hbm, ybuf, bitmap,
               xb0, xb1, ob0, ob1, isem0, isem1, osem0, osem1):
    wid = lax.axis_index("s") * _NC + lax.axis_index("c")
    base = wid * _PER_TILE
    obase = wid * (_PER_TILE // 4)

    xrow = wid * (_PER_TILE // 4096)
    pltpu.make_async_copy(x_hbm.at[pl.ds(xrow, 8), :], xb0, isem0).start()
    pltpu.make_async_copy(x_hbm.at[pl.ds(xrow + 8, 8), :], xb1, isem1).start()

    pltpu.sync_copy(y_hbm, ybuf)

    zeros = jnp.zeros((16,), jnp.int32)

    def zbody(i, c):
        bitmap[pl.ds(i * 16, 16)] = zeros
        return c

    lax.fori_loop(0, _NWORDS // 16, zbody, 0, unroll=4)

    iota4 = lax.broadcasted_iota(jnp.int32, (16,), 0) * 4
    one = jnp.full((16,), 1, jnp.int32)

    def bbody(i, c):
        yv = ybuf[pl.ds(i * 16, 16)]
        w = lax.shift_right_logical(yv, 5)
        sh = yv & 31
        bit = one << sh

        def round_fn(_, c2):
            cur = plsc.load_gather(bitmap, [w])
            miss = (lax.shift_right_logical(cur, sh) & one) == 0
            plsc.store_scatter(bitmap, [w], cur | bit, mask=miss)
            return c2

        round_fn(0, 0)
        cur = plsc.load_gather(bitmap, [w])
        miss = (lax.shift_right_logical(cur, sh) & one) == 0
        cnt = plsc.all_reduce_population_count(miss)

        @pl.when(cnt[0] > 0)
        def _():
            lax.fori_loop(0, 15, round_fn, 0)

        return c

    lax.fori_loop(0, _NY // 16, bbody, 0)

    def compute_chunk(xb, ob):
        @plsc.parallel_loop(0, _GROUPS, unroll=8)
        def gbody(g):
            wr = g >> 8
            c0 = (g & 255) * 16
            acc = None
            for k in range(4):
                xk = xb[wr * 4 + k, pl.ds(c0, 16)]
                word = plsc.load_gather(bitmap, [lax.shift_right_logical(xk, 5)])
                r = lax.shift_right_logical(word, xk & 31) & one
                acc = r if k == 0 else acc | (r << (8 * k))
            ob[wr, pl.ds(c0, 16)] = acc

    bufs = ((xb0, ob0, isem0, osem0), (xb1, ob1, isem1, osem1))
    _ROWS = _CH // 4096
    row_base = wid * (_PER_TILE // 4096)

    def pair_body(p, c):
        for b in range(2):
            xb, ob, isem, osem = bufs[b]
            ci = p * 2 + b
            pltpu.make_async_copy(
                x_hbm.at[pl.ds(xrow + ci * 8, 8), :], xb, isem).wait()

            @pl.when(ci >= 2)
            def _():
                prev = row_base + (ci - 2) * _ROWS
                pltpu.make_async_copy(
                    ob.bitcast(jnp.int8),
                    out_hbm.at[pl.ds(prev, _ROWS), :], osem).wait()

            compute_chunk(xb, ob)
            pltpu.make_async_copy(
                ob.bitcast(jnp.int8),
                out_hbm.at[pl.ds(row_base + ci * _ROWS, _ROWS), :],
                osem).start()

            @pl.when(p < _PAIRS - 1)
            def _():
                nxt = xrow + (ci + 2) * 8
                pltpu.make_async_copy(
                    x_hbm.at[pl.ds(nxt, 8), :], xb, isem).start()
        return c

    lax.fori_loop(0, _PAIRS, pair_body, 0)

    for b in range(2):
        xb, ob, isem, osem = bufs[b]
        pltpu.make_async_copy(
            ob.bitcast(jnp.int8),
            out_hbm.at[pl.ds(row_base + (_CHUNKS - 2 + b) * _ROWS, _ROWS), :],
            osem).wait()


_isin_sc = functools.partial(
    pl.kernel,
    out_type=jax.ShapeDtypeStruct((8192, 4096), jnp.int8),
    mesh=plsc.VectorSubcoreMesh(core_axis_name="c", subcore_axis_name="s"),
    scratch_types=[
        pltpu.VMEM((_NY,), jnp.int32),
        pltpu.VMEM((_NWORDS,), jnp.int32),
        pltpu.VMEM((8, 4096), jnp.int32),
        pltpu.VMEM((8, 4096), jnp.int32),
        pltpu.VMEM((_CH // 16384, 4096), jnp.int32),
        pltpu.VMEM((_CH // 16384, 4096), jnp.int32),
        pltpu.SemaphoreType.DMA,
        pltpu.SemaphoreType.DMA,
        pltpu.SemaphoreType.DMA,
        pltpu.SemaphoreType.DMA,
    ],
    compiler_params=pltpu.CompilerParams(needs_layout_passes=False),
)(_isin_body)


def kernel(x, y):
    return _isin_sc(x.astype(jnp.int32), y.astype(jnp.int32)).astype(jnp.bool_)


# speedup vs baseline: 2.7229x; 1.0006x over previous
import functools

import jax
import jax.numpy as jnp
from jax import lax
from jax.experimental import pallas as pl
from jax.experimental.pallas import tpu as pltpu
from jax.experimental.pallas import tpu_sc as plsc

_NC = 2
_NS = 16
_NW = _NC * _NS

_N = 8192 * 4096
_PER_TILE = _N // _NW
_CH = 32768
_CHUNKS = _PER_TILE // _CH
_PAIRS = _CHUNKS // 2
_GROUPS = _CH // 64
_NY = 4096
_NWORDS = 1 << 15


def _isin_body(x_hbm, y_hbm, out_hbm, ybuf, bitmap,
               xb0, xb1, ob0, ob1, isem0, isem1, osem0, osem1):
    wid = lax.axis_index("s") * _NC + lax.axis_index("c")

    xrow = wid * (_PER_TILE // 4096)
    pltpu.make_async_copy(x_hbm.at[pl.ds(xrow, 8), :], xb0, isem0).start()
    pltpu.make_async_copy(x_hbm.at[pl.ds(xrow + 8, 8), :], xb1, isem1).start()

    pltpu.sync_copy(y_hbm, ybuf)

    zeros = jnp.zeros((16,), jnp.int32)

    def zbody(i, c):
        bitmap[pl.ds(i * 16, 16)] = zeros
        return c

    lax.fori_loop(0, _NWORDS // 16, zbody, 0, unroll=4)

    one = jnp.full((16,), 1, jnp.int32)

    def bbody(i, c):
        yv = ybuf[pl.ds(i * 16, 16)]
        w = lax.shift_right_logical(yv, 5)
        sh = yv & 31
        bit = one << sh

        def round_fn(_, c2):
            cur = plsc.load_gather(bitmap, [w])
            miss = (lax.shift_right_logical(cur, sh) & one) == 0
            plsc.store_scatter(bitmap, [w], cur | bit, mask=miss)
            return c2

        round_fn(0, 0)
        cur = plsc.load_gather(bitmap, [w])
        miss = (lax.shift_right_logical(cur, sh) & one) == 0
        cnt = plsc.all_reduce_population_count(miss)

        @pl.when(cnt[0] > 0)
        def _():
            lax.fori_loop(0, 15, round_fn, 0)

        return c

    lax.fori_loop(0, _NY // 16, bbody, 0)

    def compute_chunk(xb, ob):
        @plsc.parallel_loop(0, _GROUPS, unroll=8)
        def gbody(g):
            wr = g >> 8
            c0 = (g & 255) * 16
            acc = None
            for k in range(4):
                xk = xb[wr * 4 + k, pl.ds(c0, 16)]
                word = plsc.load_gather(bitmap, [lax.shift_right_logical(xk, 5)])
                r = lax.shift_right_logical(word, xk & 31) & one
                acc = r if k == 0 else acc | (r << (8 * k))
            ob[wr, pl.ds(c0, 16)] = acc

    bufs = ((xb0, ob0, isem0, osem0), (xb1, ob1, isem1, osem1))
    _ROWS = _CH // 4096
    row_base = wid * (_PER_TILE // 4096)

    def pair_body(p, c):
        for b in range(2):
            xb, ob, isem, osem = bufs[b]
            ci = p * 2 + b
            pltpu.make_async_copy(
                x_hbm.at[pl.ds(xrow + ci * 8, 8), :], xb, isem).wait()

            @pl.when(ci >= 2)
            def _():
                prev = row_base + (ci - 2) * _ROWS
                pltpu.make_async_copy(
                    ob.bitcast(jnp.int8),
                    out_hbm.at[pl.ds(prev, _ROWS), :], osem).wait()

            compute_chunk(xb, ob)
            pltpu.make_async_copy(
                ob.bitcast(jnp.int8),
                out_hbm.at[pl.ds(row_base + ci * _ROWS, _ROWS), :],
                osem).start()

            @pl.when(p < _PAIRS - 1)
            def _():
                nxt = xrow + (ci + 2) * 8
                pltpu.make_async_copy(
                    x_hbm.at[pl.ds(nxt, 8), :], xb, isem).start()
        return c

    lax.fori_loop(0, _PAIRS, pair_body, 0)

    for b in range(2):
        xb, ob, isem, osem = bufs[b]
        pltpu.make_async_copy(
            ob.bitcast(jnp.int8),
            out_hbm.at[pl.ds(row_base + (_CHUNKS - 2 + b) * _ROWS, _ROWS), :],
            osem).wait()


_isin_sc = functools.partial(
    pl.kernel,
    out_type=jax.ShapeDtypeStruct((8192, 4096), jnp.int8),
    mesh=plsc.VectorSubcoreMesh(core_axis_name="c", subcore_axis_name="s"),
    scratch_types=[
        pltpu.VMEM((_NY,), jnp.int32),
        pltpu.VMEM((_NWORDS,), jnp.int32),
        pltpu.VMEM((8, 4096), jnp.int32),
        pltpu.VMEM((8, 4096), jnp.int32),
        pltpu.VMEM((_CH // 16384, 4096), jnp.int32),
        pltpu.VMEM((_CH // 16384, 4096), jnp.int32),
        pltpu.SemaphoreType.DMA,
        pltpu.SemaphoreType.DMA,
        pltpu.SemaphoreType.DMA,
        pltpu.SemaphoreType.DMA,
    ],
    compiler_params=pltpu.CompilerParams(needs_layout_passes=False),
)(_isin_body)


def kernel(x, y):
    return _isin_sc(x.astype(jnp.int32), y.astype(jnp.int32)).astype(jnp.bool_)
